# Initial kernel scaffold; baseline (speedup 1.0000x reference)
import jax
import jax.numpy as jnp
from jax.experimental import pallas as pl


def kernel(input, weight):
    raise NotImplementedError("write your pallas kernel here")


# speedup vs baseline: 1.5760x; 1.5760x over previous
import functools

import jax
import jax.numpy as jnp
from jax import lax
from jax.experimental import pallas as pl
from jax.experimental.pallas import tpu as pltpu
from jax.experimental.pallas import tpu_sc as plsc

_NC = 2
_NS = 16
_NW = _NC * _NS

_CHUNK = 128


@functools.lru_cache(maxsize=None)
def _make_gather(B: int, V: int, D: int):
    assert B % (_NW * _CHUNK) == 0
    b_per_w = B // _NW
    n_chunks = b_per_w // _CHUNK

    mesh = plsc.VectorSubcoreMesh(
        core_axis_name="c", subcore_axis_name="s",
        num_cores=_NC, num_subcores=_NS,
    )

    @functools.partial(
        pl.kernel,
        out_type=jax.ShapeDtypeStruct((B, D), jnp.float32),
        mesh=mesh,
        scratch_types=[
            pltpu.VMEM((_CHUNK,), jnp.int32),
            pltpu.VMEM((_CHUNK, D), jnp.float32),
            pltpu.SemaphoreType.DMA,
        ],
        compiler_params=pltpu.CompilerParams(use_tc_tiling_on_sc=False),
    )
    def gather_kernel(idx_hbm, table_hbm, out_hbm, idx_v, rows_v, sem):
        wid = lax.axis_index("s") * _NC + lax.axis_index("c")
        base = wid * b_per_w

        def step(i, carry):
            off = base + i * _CHUNK
            pltpu.sync_copy(idx_hbm.at[pl.ds(off, _CHUNK)], idx_v)
            pltpu.async_copy(table_hbm.at[idx_v], rows_v, sem).wait()
            pltpu.sync_copy(rows_v, out_hbm.at[pl.ds(off, _CHUNK)])
            return carry

        lax.fori_loop(0, n_chunks, step, 0)

    return gather_kernel


def kernel(input, weight):
    bsz, hist = input.shape
    V, D = weight.shape
    idx = input.reshape(-1).astype(jnp.int32)
    out = _make_gather(idx.shape[0], V, D)(idx, weight)
    return out.reshape(bsz, hist, D)


# speedup vs baseline: 1.8777x; 1.1915x over previous
import functools

import jax
import jax.numpy as jnp
from jax import lax
from jax.experimental import pallas as pl
from jax.experimental.pallas import tpu as pltpu
from jax.experimental.pallas import tpu_sc as plsc

_NC = 2
_NS = 16
_NW = _NC * _NS

_CHUNK = 128
_SUP = 2
_ROWS = _CHUNK * _SUP
_NBUF = 4


@functools.lru_cache(maxsize=None)
def _make_gather(B: int, V: int, D: int):
    assert B % (_NW * _ROWS * _NBUF) == 0
    b_per_w = B // _NW
    n_sup = b_per_w // _ROWS
    assert n_sup % _NBUF == 0 and n_sup >= 2 * _NBUF

    mesh = plsc.VectorSubcoreMesh(
        core_axis_name="c", subcore_axis_name="s",
        num_cores=_NC, num_subcores=_NS,
    )

    @functools.partial(
        pl.kernel,
        out_type=jax.ShapeDtypeStruct((B, D), jnp.float32),
        mesh=mesh,
        scratch_types=[
            pltpu.VMEM((b_per_w,), jnp.int32),
            [pltpu.VMEM((_ROWS, D), jnp.float32) for _ in range(_NBUF)],
            [pltpu.SemaphoreType.DMA for _ in range(_NBUF)],
            [pltpu.SemaphoreType.DMA for _ in range(_NBUF)],
        ],
        compiler_params=pltpu.CompilerParams(use_tc_tiling_on_sc=False),
    )
    def gather_kernel(idx_hbm, table_hbm, out_hbm, idx_v, rows, sem_g, sem_o):
        wid = lax.axis_index("s") * _NC + lax.axis_index("c")
        base = wid * b_per_w

        pltpu.sync_copy(idx_hbm.at[pl.ds(base, b_per_w)], idx_v)

        def gather_descs(s, b):
            return [
                pltpu.make_async_copy(
                    table_hbm.at[idx_v.at[pl.ds(s * _ROWS + c * _CHUNK, _CHUNK)]],
                    rows[b].at[pl.ds(c * _CHUNK, _CHUNK)],
                    sem_g[b],
                )
                for c in range(_SUP)
            ]

        def out_desc(s, b):
            return pltpu.make_async_copy(
                rows[b], out_hbm.at[pl.ds(base + s * _ROWS, _ROWS)], sem_o[b]
            )

        def issue_gathers(s, b):
            for d in gather_descs(s, b):
                d.start()

        def wait_gathers(s, b):
            for d in gather_descs(s, b):
                d.wait()

        issue_gathers(0, 0)
        issue_gathers(1, 1)
        wait_gathers(0, 0)
        out_desc(0, 0).start()
        issue_gathers(2, 2)
        wait_gathers(1, 1)
        out_desc(1, 1).start()
        issue_gathers(3, 3)

        def block(g, carry):
            for b in range(_NBUF):
                s = _NBUF + g * _NBUF + b
                out_desc(s - _NBUF, b).wait()
                issue_gathers(s, b)
                bq = (b + _NBUF - 2) % _NBUF
                wait_gathers(s - 2, bq)
                out_desc(s - 2, bq).start()
            return carry

        lax.fori_loop(0, n_sup // _NBUF - 1, block, 0)

        for s in (n_sup - 2, n_sup - 1):
            b = s % _NBUF
            wait_gathers(s, b)
            out_desc(s, b).start()
        for s in range(n_sup - _NBUF, n_sup):
            out_desc(s, s % _NBUF).wait()

    return gather_kernel


def kernel(input, weight):
    bsz, hist = input.shape
    V, D = weight.shape
    idx = input.reshape(-1).astype(jnp.int32)
    out = _make_gather(idx.shape[0], V, D)(idx, weight)
    return out.reshape(bsz, hist, D)
